# baseline (device time: 41509 ns/iter reference)
import jax
import jax.numpy as jnp
from jax import lax
from jax.experimental import pallas as pl
from jax.experimental.pallas import tpu as pltpu

T = 2048
T_HALF = T // 2
D = 1024
V_SHARD = 16384

K = 8
C = T_HALF // K


def kernel(ids, E):
    my_x = lax.axis_index("x")
    my_y = lax.axis_index("y")

    ids_half = lax.dynamic_slice(ids, (my_x * T_HALF,), (T_HALF,))
    local = ids_half - my_y * V_SHARD
    ok = (local >= 0) & (local < V_SHARD)
    rows = jnp.where(ok, local, 0).astype(jnp.int32)
    okf = ok.astype(jnp.float32).reshape(T_HALF, 1)

    def body(rows_ref, okf_ref, e_ref, out_ref,
             g_ref, p_ref, recv_y,
             gsem, ysend, yrecv, xsend, xrecv):
        mx = lax.axis_index("x")
        my = lax.axis_index("y")

        barrier = pltpu.get_barrier_semaphore()
        pl.semaphore_signal(barrier, inc=1, device_id=(mx, 1 - my),
                            device_id_type=pl.DeviceIdType.MESH)
        pl.semaphore_signal(barrier, inc=1, device_id=(1 - mx, my),
                            device_id_type=pl.DeviceIdType.MESH)
        pl.semaphore_wait(barrier, 2)

        def chunk(ref, c):
            return ref.at[pl.ds(c * C, C), :]

        def my_half(c):
            return out_ref.at[pl.ds(mx * T_HALF + c * C, C), :]

        def other_half(c):
            return out_ref.at[pl.ds((1 - mx) * T_HALF + c * C, C), :]

        def issue_gather(c, sem):
            for i in range(C):
                t = c * C + i
                row = rows_ref[t]
                pltpu.make_async_copy(
                    e_ref.at[pl.ds(row, 1), :],
                    g_ref.at[pl.ds(t, 1), :],
                    sem,
                ).start()

        rdma_y = []
        rdma_x = []

        def reduce_and_forward(j):
            rdma_y[j].wait_recv()
            my_half(j)[...] = p_ref[pl.ds(j * C, C), :] + recv_y[pl.ds(j * C, C), :]
            rx = pltpu.make_async_remote_copy(
                src_ref=my_half(j),
                dst_ref=my_half(j),
                send_sem=xsend.at[j],
                recv_sem=xrecv.at[j],
                device_id=(1 - mx, my),
                device_id_type=pl.DeviceIdType.MESH,
            )
            rx.start()
            rdma_x.append(rx)

        for c0 in range(4):
            issue_gather(c0, gsem.at[c0 % 4])

        for c in range(K):
            pltpu.make_async_copy(
                e_ref.at[pl.ds(0, C), :], chunk(g_ref, c), gsem.at[c % 4]
            ).wait()
            p_ref[pl.ds(c * C, C), :] = (
                g_ref[pl.ds(c * C, C), :] * okf_ref[pl.ds(c * C, C), :]
            ).astype(jnp.bfloat16)

            ry = pltpu.make_async_remote_copy(
                src_ref=chunk(p_ref, c),
                dst_ref=chunk(recv_y, c),
                send_sem=ysend.at[c],
                recv_sem=yrecv.at[c],
                device_id=(mx, 1 - my),
                device_id_type=pl.DeviceIdType.MESH,
            )
            ry.start()
            rdma_y.append(ry)

            if c + 4 < K:
                issue_gather(c + 4, gsem.at[c % 4])

            if c >= 1:
                reduce_and_forward(c - 1)
        reduce_and_forward(K - 1)

        for j in range(K):
            rrecv = pltpu.make_async_remote_copy(
                src_ref=other_half(j),
                dst_ref=other_half(j),
                send_sem=xsend.at[j],
                recv_sem=xrecv.at[j],
                device_id=(1 - mx, my),
                device_id_type=pl.DeviceIdType.MESH,
            )
            rrecv.wait_recv()
        for j in range(K):
            rdma_y[j].wait_send()
            rdma_x[j].wait_send()

    return pl.pallas_call(
        body,
        out_shape=jax.ShapeDtypeStruct((T, D), jnp.bfloat16),
        in_specs=[
            pl.BlockSpec(memory_space=pltpu.SMEM),
            pl.BlockSpec(memory_space=pltpu.VMEM),
            pl.BlockSpec(memory_space=pltpu.HBM),
        ],
        out_specs=pl.BlockSpec(memory_space=pltpu.VMEM),
        scratch_shapes=[
            pltpu.VMEM((T_HALF, D), jnp.float32),
            pltpu.VMEM((T_HALF, D), jnp.bfloat16),
            pltpu.VMEM((T_HALF, D), jnp.bfloat16),
            pltpu.SemaphoreType.DMA((4,)),
            pltpu.SemaphoreType.DMA((K,)),
            pltpu.SemaphoreType.DMA((K,)),
            pltpu.SemaphoreType.DMA((K,)),
            pltpu.SemaphoreType.DMA((K,)),
        ],
        compiler_params=pltpu.CompilerParams(collective_id=0),
    )(rows, okf, E)


# device time: 40221 ns/iter; 1.0320x vs baseline; 1.0320x over previous
import jax
import jax.numpy as jnp
from jax import lax
from jax.experimental import pallas as pl
from jax.experimental.pallas import tpu as pltpu

T = 2048
T_HALF = T // 2
D = 1024
V_SHARD = 16384

K = 8
C = T_HALF // K


def kernel(ids, E):
    my_x = lax.axis_index("x")
    my_y = lax.axis_index("y")

    ids_half = lax.dynamic_slice(ids, (my_x * T_HALF,), (T_HALF,))
    local = ids_half - my_y * V_SHARD
    ok = (local >= 0) & (local < V_SHARD)
    rows = jnp.where(ok, local, 0).astype(jnp.int32)
    okf = ok.astype(jnp.float32).reshape(T_HALF, 1)

    def body(rows_ref, okf_ref, e_ref, out_ref,
             g_ref, p_ref, recv_y,
             gsem, ysend, yrecv, xsend, xrecv):
        mx = lax.axis_index("x")
        my = lax.axis_index("y")

        barrier = pltpu.get_barrier_semaphore()
        pl.semaphore_signal(barrier, inc=1, device_id=(mx, 1 - my),
                            device_id_type=pl.DeviceIdType.MESH)
        pl.semaphore_signal(barrier, inc=1, device_id=(1 - mx, my),
                            device_id_type=pl.DeviceIdType.MESH)
        pl.semaphore_wait(barrier, 2)

        def chunk(ref, c):
            return ref.at[pl.ds(c * C, C), :]

        def my_half(c):
            return out_ref.at[pl.ds(mx * T_HALF + c * C, C), :]

        def other_half(c):
            return out_ref.at[pl.ds((1 - mx) * T_HALF + c * C, C), :]

        def issue_gather(c, sem):
            for i in range(C):
                t = c * C + i
                row = rows_ref[t]
                pltpu.make_async_copy(
                    e_ref.at[pl.ds(row, 1), :],
                    g_ref.at[pl.ds(t, 1), :],
                    sem,
                ).start()

        rdma_y = []
        rdma_x = []

        def reduce_and_forward(j):
            rdma_y[j].wait_recv()
            my_half(j)[...] = p_ref[pl.ds(j * C, C), :] + recv_y[pl.ds(j * C, C), :]
            rx = pltpu.make_async_remote_copy(
                src_ref=my_half(j),
                dst_ref=my_half(j),
                send_sem=xsend.at[j],
                recv_sem=xrecv.at[j],
                device_id=(1 - mx, my),
                device_id_type=pl.DeviceIdType.MESH,
            )
            rx.start()
            rdma_x.append(rx)

        issue_gather(0, gsem.at[0])
        issue_gather(1, gsem.at[1])

        for c in range(K):
            pltpu.make_async_copy(
                e_ref.at[pl.ds(0, C), :], chunk(g_ref, c), gsem.at[c % 2]
            ).wait()
            p_ref[pl.ds(c * C, C), :] = (
                g_ref[pl.ds(c * C, C), :] * okf_ref[pl.ds(c * C, C), :]
            ).astype(jnp.bfloat16)

            ry = pltpu.make_async_remote_copy(
                src_ref=chunk(p_ref, c),
                dst_ref=chunk(recv_y, c),
                send_sem=ysend.at[c],
                recv_sem=yrecv.at[c],
                device_id=(mx, 1 - my),
                device_id_type=pl.DeviceIdType.MESH,
            )
            ry.start()
            rdma_y.append(ry)

            if c + 2 < K:
                issue_gather(c + 2, gsem.at[c % 2])

            if c >= 1:
                reduce_and_forward(c - 1)
        reduce_and_forward(K - 1)

        for j in range(K):
            rrecv = pltpu.make_async_remote_copy(
                src_ref=other_half(j),
                dst_ref=other_half(j),
                send_sem=xsend.at[j],
                recv_sem=xrecv.at[j],
                device_id=(1 - mx, my),
                device_id_type=pl.DeviceIdType.MESH,
            )
            rrecv.wait_recv()
        for j in range(K):
            rdma_y[j].wait_send()
            rdma_x[j].wait_send()

    return pl.pallas_call(
        body,
        out_shape=jax.ShapeDtypeStruct((T, D), jnp.bfloat16),
        in_specs=[
            pl.BlockSpec(memory_space=pltpu.SMEM),
            pl.BlockSpec(memory_space=pltpu.VMEM),
            pl.BlockSpec(memory_space=pltpu.HBM),
        ],
        out_specs=pl.BlockSpec(memory_space=pltpu.VMEM),
        scratch_shapes=[
            pltpu.VMEM((T_HALF, D), jnp.float32),
            pltpu.VMEM((T_HALF, D), jnp.bfloat16),
            pltpu.VMEM((T_HALF, D), jnp.bfloat16),
            pltpu.SemaphoreType.DMA((2,)),
            pltpu.SemaphoreType.DMA((K,)),
            pltpu.SemaphoreType.DMA((K,)),
            pltpu.SemaphoreType.DMA((K,)),
            pltpu.SemaphoreType.DMA((K,)),
        ],
        compiler_params=pltpu.CompilerParams(collective_id=0),
    )(rows, okf, E)
